# baseline (speedup 1.0000x reference)
import jax
import jax.numpy as jnp
from jax.experimental import pallas as pl
from jax.experimental.pallas import tpu as pltpu

N = 50000
D = 512
H = 64
B = 1024
BN = 8192
NB = (N + BN - 1) // BN
SB = 128
NW = B // SB
ACC_R = B + SB


def _pool_kernel(pre_ref, x_ref, ids_ref, w1t_ref, b1_ref, w2_ref, b2_ref,
                 out_ref, acc_ref, den_ref):
    i = pl.program_id(0)

    @pl.when(i == 0)
    def _init():
        acc_ref[...] = jnp.zeros_like(acc_ref)
        den_ref[...] = jnp.zeros_like(den_ref)

    x = x_ref[...]
    valid = (jax.lax.broadcasted_iota(jnp.int32, (BN, 1), 0) + i * BN) < N
    xb = jnp.where(valid, x, 0.0).astype(jnp.bfloat16)
    h = jnp.tanh(
        jax.lax.dot_general(xb, w1t_ref[...],
                            (((1,), (0,)), ((), ())),
                            preferred_element_type=jnp.float32)
        + b1_ref[...])
    s = jnp.sum(h * w2_ref[...], axis=1, keepdims=True)
    s_row = jnp.transpose(s) + b2_ref[...]
    valid_row = (jax.lax.broadcasted_iota(jnp.int32, (1, BN), 1)
                 + i * BN) < N
    e_row = jnp.where(valid_row, jnp.exp(s_row), 0.0)

    ids = ids_ref[0]
    base = pre_ref[0, i]
    base_al = (base // 8) * 8
    max_rel = pre_ref[1, i] - base_al

    def window(w):
        rel = ids - (base_al + w * SB)
        match = jax.lax.broadcasted_iota(jnp.int32, (SB, BN), 0) == rel
        wmatch = jnp.where(match, e_row, 0.0)
        partial = jax.lax.dot_general(
            wmatch.astype(jnp.bfloat16), xb, (((1,), (0,)), ((), ())),
            preferred_element_type=jnp.float32)
        dsum = jnp.sum(wmatch, axis=1, keepdims=True)
        start = base_al + w * SB
        acc_ref[pl.ds(start, SB), :] += partial
        den_ref[pl.ds(start, SB), :] += dsum

    window(0)
    for w in range(1, NW):
        @pl.when(max_rel >= w * SB)
        def _w(w=w):
            window(w)

    @pl.when(i == NB - 1)
    def _finish():
        out_ref[...] = acc_ref[:B, :] / (den_ref[:B, :] + 1e-16)


@jax.jit
def kernel(x, W1, b1, W2, b2, batch):
    ids = batch.astype(jnp.int32)
    ids_p = jnp.pad(ids, (0, NB * BN - N), constant_values=B)
    ids3 = ids_p.reshape(NB, 1, BN)
    pre = jnp.stack([ids_p[::BN], ids_p[BN - 1::BN]])

    grid_spec = pltpu.PrefetchScalarGridSpec(
        num_scalar_prefetch=1,
        grid=(NB,),
        in_specs=[
            pl.BlockSpec((BN, D), lambda i, pre: (i, 0)),
            pl.BlockSpec((1, 1, BN), lambda i, pre: (i, 0, 0)),
            pl.BlockSpec((D, H), lambda i, pre: (0, 0)),
            pl.BlockSpec((1, H), lambda i, pre: (0, 0)),
            pl.BlockSpec((1, H), lambda i, pre: (0, 0)),
            pl.BlockSpec((1, 1), lambda i, pre: (0, 0)),
        ],
        out_specs=pl.BlockSpec((B, D), lambda i, pre: (0, 0)),
        scratch_shapes=[
            pltpu.VMEM((ACC_R, D), jnp.float32),
            pltpu.VMEM((ACC_R, 1), jnp.float32),
        ],
    )
    out = pl.pallas_call(
        _pool_kernel,
        grid_spec=grid_spec,
        out_shape=jax.ShapeDtypeStruct((B, D), jnp.float32),
        compiler_params=pltpu.CompilerParams(
            dimension_semantics=("arbitrary",)),
    )(pre, x, ids3, W1.T.astype(jnp.bfloat16), b1.reshape(1, H),
      W2.reshape(1, H), b2.reshape(1, 1))
    return out


# speedup vs baseline: 1.2118x; 1.2118x over previous
import jax
import jax.numpy as jnp
from jax.experimental import pallas as pl
from jax.experimental.pallas import tpu as pltpu

N = 50000
D = 512
H = 64
B = 1024
BN = 4096
NB = (N + BN - 1) // BN
SB = 128
NW = B // SB
ACC_R = B + SB


def _pool_kernel(pre_ref, x_ref, ids_ref, w1t_ref, b1_ref, w2_ref, b2_ref,
                 out_ref, acc_ref, den_ref, xb_ref, er_ref):
    i = pl.program_id(0)
    par = jax.lax.rem(i, 2)

    @pl.when(i == 0)
    def _init():
        acc_ref[...] = jnp.zeros_like(acc_ref)
        den_ref[...] = jnp.zeros_like(den_ref)

    @pl.when(i < NB)
    def _mlp():
        x = x_ref[...]
        valid = (jax.lax.broadcasted_iota(jnp.int32, (BN, 1), 0)
                 + i * BN) < N
        xb = jnp.where(valid, x, 0.0).astype(jnp.bfloat16)
        h = jnp.tanh(
            jax.lax.dot_general(xb, w1t_ref[...], (((1,), (0,)), ((), ())),
                                preferred_element_type=jnp.float32)
            + b1_ref[...])
        s = jnp.sum(h * w2_ref[...], axis=1, keepdims=True)
        s_row = jnp.transpose(s) + b2_ref[...]
        valid_row = (jax.lax.broadcasted_iota(jnp.int32, (1, BN), 1)
                     + i * BN) < N
        e_row = jnp.where(valid_row, jnp.exp(s_row), 0.0)
        xb_ref[pl.ds(par, 1)] = xb[None]
        er_ref[pl.ds(par, 1)] = e_row[None]

    @pl.when(i > 0)
    def _pool():
        j = i - 1
        xb = xb_ref[1 - par]
        e_row = er_ref[1 - par]
        ids = ids_ref[0]
        base = pre_ref[0, j]
        base_al = (base // 8) * 8
        max_rel = pre_ref[1, j] - base_al

        def window(w):
            rel = ids - (base_al + w * SB)
            match = jax.lax.broadcasted_iota(jnp.int32, (SB, BN), 0) == rel
            wmatch = jnp.where(match, e_row, 0.0)
            partial = jax.lax.dot_general(
                wmatch.astype(jnp.bfloat16), xb, (((1,), (0,)), ((), ())),
                preferred_element_type=jnp.float32)
            dsum = jnp.sum(wmatch, axis=1, keepdims=True)
            start = base_al + w * SB
            acc_ref[pl.ds(start, SB), :] += partial
            den_ref[pl.ds(start, SB), :] += dsum

        window(0)
        for w in range(1, NW):
            @pl.when(max_rel >= w * SB)
            def _w(w=w):
                window(w)

    @pl.when(i == NB)
    def _finish():
        out_ref[...] = acc_ref[:B, :] / (den_ref[:B, :] + 1e-16)


@jax.jit
def kernel(x, W1, b1, W2, b2, batch):
    ids = batch.astype(jnp.int32)
    ids_p = jnp.pad(ids, (0, NB * BN - N), constant_values=B)
    ids3 = ids_p.reshape(NB, 1, BN)
    pre = jnp.stack([ids_p[::BN], ids_p[BN - 1::BN]])

    grid_spec = pltpu.PrefetchScalarGridSpec(
        num_scalar_prefetch=1,
        grid=(NB + 1,),
        in_specs=[
            pl.BlockSpec((BN, D),
                         lambda i, pre: (jnp.minimum(i, NB - 1), 0)),
            pl.BlockSpec((1, 1, BN),
                         lambda i, pre: (jnp.maximum(i - 1, 0), 0, 0)),
            pl.BlockSpec((D, H), lambda i, pre: (0, 0)),
            pl.BlockSpec((1, H), lambda i, pre: (0, 0)),
            pl.BlockSpec((1, H), lambda i, pre: (0, 0)),
            pl.BlockSpec((1, 1), lambda i, pre: (0, 0)),
        ],
        out_specs=pl.BlockSpec((B, D), lambda i, pre: (0, 0)),
        scratch_shapes=[
            pltpu.VMEM((ACC_R, D), jnp.float32),
            pltpu.VMEM((ACC_R, 1), jnp.float32),
            pltpu.VMEM((2, BN, D), jnp.bfloat16),
            pltpu.VMEM((2, 1, BN), jnp.float32),
        ],
    )
    out = pl.pallas_call(
        _pool_kernel,
        grid_spec=grid_spec,
        out_shape=jax.ShapeDtypeStruct((B, D), jnp.float32),
        compiler_params=pltpu.CompilerParams(
            dimension_semantics=("arbitrary",)),
    )(pre, x, ids3, W1.T.astype(jnp.bfloat16), b1.reshape(1, H),
      W2.reshape(1, H), b2.reshape(1, 1))
    return out
